# Initial kernel scaffold; baseline (speedup 1.0000x reference)
import jax
import jax.numpy as jnp
from jax.experimental import pallas as pl


def kernel(x, pos, edge_index, W_local, b_local, W_global, b_global):
    raise NotImplementedError("write your pallas kernel here")


# speedup vs baseline: 1.7495x; 1.7495x over previous
import functools

import jax
import jax.numpy as jnp
from jax import lax
from jax.experimental import pallas as pl
from jax.experimental.pallas import tpu as pltpu
from jax.experimental.pallas import tpu_sc as plsc

NW = 32
LANES = 16
C_EDGES = 2000
K_GATHER = 32


def _mm_xs_body(xb, pb, wx, wv, xs_out, v_out):
    v = jnp.dot(pb[...], wv[...], preferred_element_type=jnp.float32)
    xs_out[...] = jnp.dot(xb[...], wx[...], preferred_element_type=jnp.float32) + v
    v_out[...] = v


def _mm_out_body(mb, vb, blb, wg, bgb, ob):
    h = jnp.maximum(mb[...] - vb[...] + blb[...], 0.0)
    ob[...] = jnp.dot(h, wg[...], preferred_element_type=jnp.float32) + bgb[...]


def _sc_segmax(np_rows, d, ep, r):
    n_chunks = ep // C_EDGES
    n_grp = C_EDGES // LANES
    mesh = plsc.VectorSubcoreMesh(core_axis_name="c", subcore_axis_name="s")

    @functools.partial(
        pl.kernel,
        mesh=mesh,
        compiler_params=pltpu.CompilerParams(needs_layout_passes=False),
        out_type=jax.ShapeDtypeStruct((np_rows, d), jnp.float32),
        scratch_types=[
            pltpu.VMEM((r + 1, d), jnp.float32),
            pltpu.VMEM((C_EDGES,), jnp.int32),
            pltpu.VMEM((C_EDGES,), jnp.int32),
            pltpu.VMEM((C_EDGES + 3 * LANES,), jnp.int32),
            pltpu.VMEM((C_EDGES + 3 * LANES,), jnp.int32),
            pltpu.VMEM((K_GATHER, d), jnp.float32),
            pltpu.SemaphoreType.DMA,
        ],
    )
    def seg_max(xs_hbm, dst_hbm, src_hbm, m_hbm,
                m_loc, dbuf, sbuf, slist, dloc, rows, sem):
        cid = lax.axis_index("c")
        sid = lax.axis_index("s")
        wid = sid * 2 + cid
        lo = wid * r
        pltpu.sync_copy(xs_hbm.at[pl.ds(lo, r)], m_loc.at[pl.ds(0, r)])

        def chunk_body(ci, carry):
            base = ci * C_EDGES
            pltpu.sync_copy(dst_hbm.at[pl.ds(base, C_EDGES)], dbuf)
            pltpu.sync_copy(src_hbm.at[pl.ds(base, C_EDGES)], sbuf)

            def grp(g, n):
                d16 = dbuf[pl.ds(g * LANES, LANES)]
                s16 = sbuf[pl.ds(g * LANES, LANES)]
                msk = (d16 >= lo) & (d16 < lo + r)
                plsc.store_compressed(slist.at[pl.ds(n, LANES)], s16, mask=msk)
                plsc.store_compressed(dloc.at[pl.ds(n, LANES)], d16 - lo, mask=msk)
                return n + jnp.sum(msk.astype(jnp.int32))

            n = lax.fori_loop(0, n_grp, grp, jnp.int32(0))
            slist[pl.ds(n, LANES)] = jnp.zeros((LANES,), jnp.int32)
            slist[pl.ds(n + LANES, LANES)] = jnp.zeros((LANES,), jnp.int32)
            dloc[pl.ds(n, LANES)] = jnp.full((LANES,), r, jnp.int32)
            dloc[pl.ds(n + LANES, LANES)] = jnp.full((LANES,), r, jnp.int32)
            nb = (n + K_GATHER - 1) // K_GATHER

            def blk(b, carry2):
                pltpu.async_copy(
                    xs_hbm.at[slist.at[pl.ds(b * K_GATHER, K_GATHER)]],
                    rows, sem).wait()

                def edge(i, carry3):
                    dl = dloc[pl.ds(b * K_GATHER + i, LANES)][0]
                    for j in range(d // LANES):
                        sl = pl.ds(j * LANES, LANES)
                        m_loc[dl, sl] = jnp.maximum(m_loc[dl, sl], rows[i, sl])
                    return carry3

                lax.fori_loop(0, K_GATHER, edge, jnp.int32(0))
                return carry2

            lax.fori_loop(0, nb, blk, jnp.int32(0))
            return carry

        lax.fori_loop(0, n_chunks, chunk_body, jnp.int32(0))
        pltpu.sync_copy(m_loc.at[pl.ds(0, r)], m_hbm.at[pl.ds(lo, r)])

    return seg_max


def kernel(x, pos, edge_index, W_local, b_local, W_global, b_global):
    n, d = x.shape
    e = edge_index.shape[1]
    r = -(-n // (NW * 8)) * 8
    np_rows = NW * r
    ep = -(-e // C_EDGES) * C_EDGES

    x_pad = jnp.zeros((np_rows, d), jnp.float32).at[:n].set(x)
    pos_pad = jnp.zeros((np_rows, 128), jnp.float32).at[:n, :3].set(pos)
    wx = W_local[:d]
    wv = jnp.zeros((128, d), jnp.float32).at[:3].set(W_local[d:])
    src = jnp.zeros((ep,), jnp.int32).at[:e].set(edge_index[0])
    dst = jnp.full((ep,), jnp.int32(1 << 30)).at[:e].set(edge_index[1])

    blk_rows = 512
    grid = (np_rows // blk_rows,)
    xs, v = pl.pallas_call(
        _mm_xs_body,
        grid=grid,
        in_specs=[
            pl.BlockSpec((blk_rows, d), lambda i: (i, 0)),
            pl.BlockSpec((blk_rows, 128), lambda i: (i, 0)),
            pl.BlockSpec((d, d), lambda i: (0, 0)),
            pl.BlockSpec((128, d), lambda i: (0, 0)),
        ],
        out_specs=[
            pl.BlockSpec((blk_rows, d), lambda i: (i, 0)),
            pl.BlockSpec((blk_rows, d), lambda i: (i, 0)),
        ],
        out_shape=[
            jax.ShapeDtypeStruct((np_rows, d), jnp.float32),
            jax.ShapeDtypeStruct((np_rows, d), jnp.float32),
        ],
    )(x_pad, pos_pad, wx, wv)

    m = _sc_segmax(np_rows, d, ep, r)(xs, dst, src)

    out = pl.pallas_call(
        _mm_out_body,
        grid=grid,
        in_specs=[
            pl.BlockSpec((blk_rows, d), lambda i: (i, 0)),
            pl.BlockSpec((blk_rows, d), lambda i: (i, 0)),
            pl.BlockSpec((1, d), lambda i: (0, 0)),
            pl.BlockSpec((d, d), lambda i: (0, 0)),
            pl.BlockSpec((1, d), lambda i: (0, 0)),
        ],
        out_specs=pl.BlockSpec((blk_rows, d), lambda i: (i, 0)),
        out_shape=jax.ShapeDtypeStruct((np_rows, d), jnp.float32),
    )(m, v, b_local.reshape(1, d), W_global, b_global.reshape(1, d))

    return out[:n]
